# Initial kernel scaffold; baseline (speedup 1.0000x reference)
import jax
import jax.numpy as jnp
from jax.experimental import pallas as pl


def kernel(input1_mean, input1_var, input2_mean, input2_var):
    raise NotImplementedError("write your pallas kernel here")


# speedup vs baseline: 3.1564x; 3.1564x over previous
import functools

import jax
import jax.numpy as jnp
from jax import lax
from jax.experimental import pallas as pl
from jax.experimental.pallas import tpu as pltpu

B = 1024
D = 128
K = 32
BLK = 128
NBLK = B // BLK


def _loss_kernel(m_full_ref, m_blk_ref, v_blk_ref, out_ref):
    i = pl.program_id(0)
    m_full = m_full_ref[...]
    m_blk = m_blk_ref[...]
    m2_full = m_full * m_full

    dot = functools.partial(
        lax.dot_general,
        preferred_element_type=jnp.float32,
        precision=lax.Precision.HIGHEST,
    )

    ones_row = jnp.ones((1, D), dtype=jnp.float32)
    rn_row = dot(ones_row, m2_full, (((1,), (1,)), ((), ())))

    g = dot(m_blk, m_full, (((1,), (1,)), ((), ())))

    score = rn_row - 2.0 * g

    col = lax.broadcasted_iota(jnp.int32, (BLK, B), 1)
    row_g = lax.broadcasted_iota(jnp.int32, (BLK, B), 0) + i * BLK
    inf = jnp.float32(jnp.inf)
    score = jnp.where(col == row_g, inf, score)

    def body(_, carry):
        score, sel = carry
        mval = jnp.min(score, axis=1, keepdims=True)
        eq = score == mval
        idx = jnp.min(jnp.where(eq, col, B), axis=1, keepdims=True)
        onehot = col == idx
        sel = sel + onehot.astype(jnp.float32)
        score = jnp.where(onehot, inf, score)
        return score, sel

    sel0 = jnp.zeros((BLK, B), dtype=jnp.float32)
    _, sel = lax.fori_loop(0, K, body, (score, sel0))

    s1 = dot(sel, m_full, (((1,), (0,)), ((), ())))
    s2 = dot(sel, m2_full, (((1,), (0,)), ((), ())))

    var = (s2 - s1 * s1 * (1.0 / K)) * (1.0 / (K - 1))
    nstd = jnp.sqrt(jnp.maximum(var, 0.0))
    pstd = jnp.exp(v_blk_ref[...] * 0.5)
    partial = jnp.sum((pstd - nstd) ** 2)

    @pl.when(i == 0)
    def _():
        out_ref[0, 0] = 0.0

    out_ref[0, 0] += partial


def kernel(input1_mean, input1_var, input2_mean, input2_var):
    del input2_mean, input2_var
    out = pl.pallas_call(
        _loss_kernel,
        grid=(NBLK,),
        in_specs=[
            pl.BlockSpec((B, D), lambda i: (0, 0)),
            pl.BlockSpec((BLK, D), lambda i: (i, 0)),
            pl.BlockSpec((BLK, D), lambda i: (i, 0)),
        ],
        out_specs=pl.BlockSpec(
            (1, 1), lambda i: (0, 0), memory_space=pltpu.SMEM
        ),
        out_shape=jax.ShapeDtypeStruct((1, 1), jnp.float32),
    )(input1_mean, input1_mean, input1_var)
    return (out[0, 0] * (1.0 / (B * D))).reshape(())


# speedup vs baseline: 3.7916x; 1.2012x over previous
import functools

import jax
import jax.numpy as jnp
from jax import lax
from jax.experimental import pallas as pl
from jax.experimental.pallas import tpu as pltpu

B = 1024
D = 128
K = 32
BLK = 128
NBLK = B // BLK


def _loss_kernel(m_full_ref, m_blk_ref, v_blk_ref, out_ref):
    i = pl.program_id(0)
    m_full = m_full_ref[...]
    m_blk = m_blk_ref[...]
    m2_full = m_full * m_full

    dot = functools.partial(
        lax.dot_general,
        preferred_element_type=jnp.float32,
        precision=lax.Precision.HIGHEST,
    )

    ones_row = jnp.ones((1, D), dtype=jnp.float32)
    rn_row = dot(ones_row, m2_full, (((1,), (1,)), ((), ())))

    g = dot(m_blk, m_full, (((1,), (1,)), ((), ())))

    score = rn_row - 2.0 * g

    col = lax.broadcasted_iota(jnp.int32, (BLK, B), 1)
    row_g = lax.broadcasted_iota(jnp.int32, (BLK, B), 0) + i * BLK

    bits = lax.bitcast_convert_type(score, jnp.int32)
    mono = bits ^ (lax.shift_right_arithmetic(bits, 31) & jnp.int32(0x7FFFFFFF))
    imax = jnp.int32(0x7FFFFFFF)
    keys = (mono & jnp.int32(~1023)) | col
    keys = jnp.where(col == row_g, imax, keys)

    def body(_, carry):
        keys, sel = carry
        kmin = jnp.min(keys, axis=1, keepdims=True)
        onehot = keys == kmin
        sel = sel + onehot.astype(jnp.float32)
        keys = jnp.where(onehot, imax, keys)
        return keys, sel

    sel0 = jnp.zeros((BLK, B), dtype=jnp.float32)
    _, sel = lax.fori_loop(0, K, body, (keys, sel0))

    s1 = dot(sel, m_full, (((1,), (0,)), ((), ())))
    s2 = dot(sel, m2_full, (((1,), (0,)), ((), ())))

    var = (s2 - s1 * s1 * (1.0 / K)) * (1.0 / (K - 1))
    nstd = jnp.sqrt(jnp.maximum(var, 0.0))
    pstd = jnp.exp(v_blk_ref[...] * 0.5)
    partial = jnp.sum((pstd - nstd) ** 2)

    @pl.when(i == 0)
    def _():
        out_ref[0, 0] = 0.0

    out_ref[0, 0] += partial


def kernel(input1_mean, input1_var, input2_mean, input2_var):
    del input2_mean, input2_var
    out = pl.pallas_call(
        _loss_kernel,
        grid=(NBLK,),
        in_specs=[
            pl.BlockSpec((B, D), lambda i: (0, 0)),
            pl.BlockSpec((BLK, D), lambda i: (i, 0)),
            pl.BlockSpec((BLK, D), lambda i: (i, 0)),
        ],
        out_specs=pl.BlockSpec(
            (1, 1), lambda i: (0, 0), memory_space=pltpu.SMEM
        ),
        out_shape=jax.ShapeDtypeStruct((1, 1), jnp.float32),
    )(input1_mean, input1_mean, input1_var)
    return (out[0, 0] * (1.0 / (B * D))).reshape(())


# speedup vs baseline: 5.3603x; 1.4137x over previous
import functools

import jax
import jax.numpy as jnp
from jax import lax
from jax.experimental import pallas as pl
from jax.experimental.pallas import tpu as pltpu

B = 1024
D = 128
K = 32
BLK = 128
NBLK = B // BLK


def _i32(x):
    x &= 0xFFFFFFFF
    return jnp.int32(x - (1 << 32) if x >= (1 << 31) else x)


def _loss_kernel(m_full_ref, m_blk_ref, v_blk_ref, out_ref):
    i = pl.program_id(0)
    m_full = m_full_ref[...]
    m_blk = m_blk_ref[...]
    m2_full = m_full * m_full

    dot = functools.partial(
        lax.dot_general,
        preferred_element_type=jnp.float32,
        precision=lax.Precision.HIGHEST,
    )

    ones_row = jnp.ones((1, D), dtype=jnp.float32)
    rn_row = dot(ones_row, m2_full, (((1,), (1,)), ((), ())))

    g = dot(m_blk, m_full, (((1,), (1,)), ((), ())))

    score = rn_row - 2.0 * g

    col = lax.broadcasted_iota(jnp.int32, (BLK, B), 1)
    row_g = lax.broadcasted_iota(jnp.int32, (BLK, B), 0) + i * BLK

    bits = lax.bitcast_convert_type(score, jnp.int32)
    mono = bits ^ (lax.shift_right_arithmetic(bits, 31) & jnp.int32(0x7FFFFFFF))
    imax = jnp.int32(0x7FFFFFFF)
    keys = (mono & jnp.int32(~1023)) | col
    keys = jnp.where(col == row_g, imax, keys)

    isign = jnp.int32(-(2**31))
    w = keys ^ isign
    ones_b1 = jnp.ones((B, 1), dtype=jnp.float32)
    p = jnp.zeros((BLK, 1), dtype=jnp.int32)
    need = jnp.full((BLK, 1), float(K), dtype=jnp.float32)
    for b in range(31, -1, -1):
        mhi = _i32(0xFFFFFFFF << b)
        eq = (w & mhi) == p
        ind = jnp.where(eq, 1.0, 0.0).astype(jnp.float32)
        c = lax.dot_general(
            ind, ones_b1, (((1,), (0,)), ((), ())),
            preferred_element_type=jnp.float32,
        )
        go1 = c < need
        p = jnp.where(go1, p | _i32(1 << b), p)
        need = jnp.where(go1, need - c, need)

    thr = p ^ isign
    sel = jnp.where(keys <= thr, 1.0, 0.0).astype(jnp.float32)

    s1 = dot(sel, m_full, (((1,), (0,)), ((), ())))
    s2 = dot(sel, m2_full, (((1,), (0,)), ((), ())))

    var = (s2 - s1 * s1 * (1.0 / K)) * (1.0 / (K - 1))
    nstd = jnp.sqrt(jnp.maximum(var, 0.0))
    pstd = jnp.exp(v_blk_ref[...] * 0.5)
    partial = jnp.sum((pstd - nstd) ** 2)

    @pl.when(i == 0)
    def _():
        out_ref[0, 0] = 0.0

    out_ref[0, 0] += partial


def kernel(input1_mean, input1_var, input2_mean, input2_var):
    del input2_mean, input2_var
    out = pl.pallas_call(
        _loss_kernel,
        grid=(NBLK,),
        in_specs=[
            pl.BlockSpec((B, D), lambda i: (0, 0)),
            pl.BlockSpec((BLK, D), lambda i: (i, 0)),
            pl.BlockSpec((BLK, D), lambda i: (i, 0)),
        ],
        out_specs=pl.BlockSpec(
            (1, 1), lambda i: (0, 0), memory_space=pltpu.SMEM
        ),
        out_shape=jax.ShapeDtypeStruct((1, 1), jnp.float32),
    )(input1_mean, input1_mean, input1_var)
    return (out[0, 0] * (1.0 / (B * D))).reshape(())


# speedup vs baseline: 6.9702x; 1.3003x over previous
import functools

import jax
import jax.numpy as jnp
from jax import lax
from jax.experimental import pallas as pl
from jax.experimental.pallas import tpu as pltpu

B = 1024
D = 128
K = 32
BLK = 128
NBLK = B // BLK


def _i32(x):
    x &= 0xFFFFFFFF
    return jnp.int32(x - (1 << 32) if x >= (1 << 31) else x)


def _loss_kernel(m_full_ref, m_blk_ref, v_blk_ref, out_ref):
    i = pl.program_id(0)
    m_full = m_full_ref[...]
    m_blk = m_blk_ref[...]
    m2_full = m_full * m_full

    dot = functools.partial(
        lax.dot_general,
        preferred_element_type=jnp.float32,
        precision=lax.Precision.HIGHEST,
    )

    ones_row = jnp.ones((1, D), dtype=jnp.float32)
    rn_row = dot(ones_row, m2_full, (((1,), (1,)), ((), ())))

    g = dot(m_blk, m_full, (((1,), (1,)), ((), ())))

    score = rn_row - 2.0 * g

    col = lax.broadcasted_iota(jnp.int32, (BLK, B), 1)
    row_g = lax.broadcasted_iota(jnp.int32, (BLK, B), 0) + i * BLK

    bits = lax.bitcast_convert_type(score, jnp.int32)
    mono = bits ^ (lax.shift_right_arithmetic(bits, 31) & jnp.int32(0x7FFFFFFF))
    imax = jnp.int32(0x7FFFFFFF)
    keys = (mono & jnp.int32(~1023)) | col
    keys = jnp.where(col == row_g, imax, keys)

    isign = jnp.int32(-(2**31))
    w = keys ^ isign
    ones_b1 = jnp.ones((B, 1), dtype=jnp.float32)
    p = jnp.zeros((BLK, 1), dtype=jnp.int32)
    need = jnp.full((BLK, 1), float(K), dtype=jnp.float32)
    for j in range(16):
        b0 = 30 - 2 * j
        mhi = _i32(0xFFFFFFFF << b0)
        wp = w & mhi
        eq0 = wp == p
        eq1 = wp == (p | _i32(1 << b0))
        eq2 = wp == (p | _i32(2 << b0))
        ind_a = jnp.where(eq0, 1.0, 0.0) + jnp.where(eq1, 4096.0, 0.0)
        ind_b = jnp.where(eq2, 1.0, 0.0)
        dot_cnt = functools.partial(
            lax.dot_general,
            dimension_numbers=(((1,), (0,)), ((), ())),
            preferred_element_type=jnp.float32,
        )
        c_a = dot_cnt(ind_a, ones_b1)
        c2 = dot_cnt(ind_b, ones_b1)
        c1 = jnp.floor(c_a * (1.0 / 4096.0))
        c0 = c_a - 4096.0 * c1
        t01 = c0 + c1
        t012 = t01 + c2
        ge1 = need > c0
        ge2 = need > t01
        ge3 = need > t012
        digit = (
            jnp.where(ge1, jnp.int32(1), jnp.int32(0))
            + jnp.where(ge2, jnp.int32(1), jnp.int32(0))
            + jnp.where(ge3, jnp.int32(1), jnp.int32(0))
        )
        p = p | (digit * _i32(1 << b0))
        need = (
            need
            - jnp.where(ge1, c0, 0.0)
            - jnp.where(ge2, c1, 0.0)
            - jnp.where(ge3, c2, 0.0)
        )

    thr = p ^ isign
    sel = jnp.where(keys <= thr, 1.0, 0.0).astype(jnp.float32)

    s1 = dot(sel, m_full, (((1,), (0,)), ((), ())))
    s2 = dot(sel, m2_full, (((1,), (0,)), ((), ())))

    var = (s2 - s1 * s1 * (1.0 / K)) * (1.0 / (K - 1))
    nstd = jnp.sqrt(jnp.maximum(var, 0.0))
    pstd = jnp.exp(v_blk_ref[...] * 0.5)
    partial = jnp.sum((pstd - nstd) ** 2)

    @pl.when(i == 0)
    def _():
        out_ref[0, 0] = 0.0

    out_ref[0, 0] += partial


def kernel(input1_mean, input1_var, input2_mean, input2_var):
    del input2_mean, input2_var
    out = pl.pallas_call(
        _loss_kernel,
        grid=(NBLK,),
        in_specs=[
            pl.BlockSpec((B, D), lambda i: (0, 0)),
            pl.BlockSpec((BLK, D), lambda i: (i, 0)),
            pl.BlockSpec((BLK, D), lambda i: (i, 0)),
        ],
        out_specs=pl.BlockSpec(
            (1, 1), lambda i: (0, 0), memory_space=pltpu.SMEM
        ),
        out_shape=jax.ShapeDtypeStruct((1, 1), jnp.float32),
    )(input1_mean, input1_mean, input1_var)
    return (out[0, 0] * (1.0 / (B * D))).reshape(())
